# baseline (speedup 1.0000x reference)
import functools

import jax
import jax.numpy as jnp
from jax import lax
from jax.experimental import pallas as pl
from jax.experimental.pallas import tpu as pltpu
from jax.experimental.pallas import tpu_sc as plsc

NSEG = 256
NROWS = 100000
D = 768
LANES = 128
CT = D // LANES
NGRP = NROWS // 8
NC, NS = 2, 16
NW = NC * NS
CHUNK = 32
GPC = CHUNK // 8
NCH = NROWS // CHUNK
NKMAX = -(-NCH // NW)
NFULL = NCH - NW * (NKMAX - 1)
NBUF = 4
CW = 16
ACC = NSEG * CT
ARS = ACC // NS
RS = NSEG // NS


def _sc_segment_sum(xp, batch2, ones, zsum, zcnt):
    mesh = plsc.VectorSubcoreMesh(
        core_axis_name="c", subcore_axis_name="s",
        num_cores=NC, num_subcores=NS)

    @functools.partial(
        pl.kernel,
        out_type=[
            jax.ShapeDtypeStruct((NC, ACC, LANES), jnp.float32),
            jax.ShapeDtypeStruct((NC, NSEG, CW), jnp.float32),
        ],
        mesh=mesh,
        scratch_types=[
            pltpu.VMEM((NKMAX, CHUNK), jnp.int32),
            pltpu.VMEM((NBUF, CT, CHUNK), jnp.int32),
            pltpu.VMEM((NBUF, CT, CHUNK, LANES), jnp.float32),
            pltpu.VMEM((CHUNK, CW), jnp.float32),
            pltpu.VMEM_SHARED((ACC, LANES), jnp.float32),
            pltpu.VMEM_SHARED((NSEG, CW), jnp.float32),
            pltpu.SemaphoreType.DMA((NBUF,)),
            pltpu.SemaphoreType.DMA((NBUF,)),
            pltpu.SemaphoreType.DMA((NBUF,)),
        ],
        compiler_params=pltpu.CompilerParams(use_tc_tiling_on_sc=False),
    )
    def body(xp_hbm, b2_hbm, ones_hbm, zs_hbm, zc_hbm,
             sums_out, cnt_out,
             idx_v, pring, bufs, ones_v, acc_s, acc_c, gsem, ssem, csem):
        c = lax.axis_index("c")
        s = lax.axis_index("s")
        wid = c * NS + s
        nk = jnp.where(wid < NFULL, NKMAX, NKMAX - 1)
        c0 = wid * (NKMAX - 1) + jnp.minimum(wid, NFULL)

        pltpu.sync_copy(zs_hbm, acc_s.at[pl.ds(s * ARS, ARS)])
        pltpu.sync_copy(zc_hbm, acc_c.at[pl.ds(s * RS, RS)])
        pltpu.sync_copy(b2_hbm.at[pl.ds(c0, NKMAX - 1)],
                        idx_v.at[pl.ds(0, NKMAX - 1)])

        @pl.when(nk == NKMAX)
        def _():
            pltpu.sync_copy(b2_hbm.at[pl.ds(c0 + NKMAX - 1, 1)],
                            idx_v.at[pl.ds(NKMAX - 1, 1)])

        pltpu.sync_copy(ones_hbm, ones_v)
        plsc.subcore_barrier()

        def start_gather(j, b):
            g0 = (c0 + j) * GPC
            for t in range(CT):
                for g in range(GPC):
                    pltpu.async_copy(xp_hbm.at[(g0 + g) * CT + t],
                                     bufs.at[b, t, pl.ds(g * 8, 8)],
                                     gsem.at[b])

        for j0 in range(NBUF - 2):
            start_gather(j0, j0)

        def wait_gather(b):
            for t in range(CT):
                for g in range(GPC):
                    pltpu.make_async_copy(xp_hbm.at[0],
                                          bufs.at[b, t, pl.ds(g * 8, 8)],
                                          gsem.at[b]).wait()

        def wait_scatter(b):
            for t in range(CT):
                pltpu.make_async_copy(bufs.at[b, t], acc_s.at[pring.at[0, 0]],
                                      ssem.at[b]).wait()
            pltpu.make_async_copy(ones_v, acc_c.at[idx_v.at[0]],
                                  csem.at[b]).wait()

        def process_chunk(k, b):
            lo = idx_v[k, pl.ds(0, 16)]
            hi = idx_v[k, pl.ds(16, 16)]
            for t in range(CT):
                pring[b, t, pl.ds(0, 16)] = lo + (t * NSEG)
                pring[b, t, pl.ds(16, 16)] = hi + (t * NSEG)
            wait_gather(b)
            for t in range(CT):
                pltpu.async_copy(bufs.at[b, t], acc_s.at[pring.at[b, t]],
                                 ssem.at[b], add=True)
            pltpu.async_copy(ones_v, acc_c.at[idx_v.at[k]], csem.at[b],
                             add=True)
            j = k + (NBUF - 2)
            bj = (b + NBUF - 2) % NBUF

            @pl.when(j < nk)
            def _():
                @pl.when(j >= NBUF)
                def _():
                    wait_scatter(bj)
                start_gather(j, bj)

        def outer_body(it, carry):
            for b in range(NBUF):
                k = it * NBUF + b

                @pl.when(k < nk)
                def _(k=k, b=b):
                    process_chunk(k, b)

            return carry

        lax.fori_loop(0, (nk + NBUF - 1) // NBUF, outer_body, 0)
        for b in range(NBUF):
            wait_scatter(b)
        plsc.subcore_barrier()

        pltpu.sync_copy(acc_s.at[pl.ds(s * ARS, ARS)],
                        sums_out.at[c, pl.ds(s * ARS, ARS)])
        pltpu.sync_copy(acc_c.at[pl.ds(s * RS, RS)],
                        cnt_out.at[c, pl.ds(s * RS, RS)])

    return body(xp, batch2, ones, zsum, zcnt)


def _tc_head(sums4, cnt2, W1, b1r, W2, b2r):
    def body(s_ref, c_ref, w1_ref, b1_ref, w2_ref, b2_ref, out_ref):
        sums = jnp.concatenate(
            [s_ref[0, t] + s_ref[1, t] for t in range(CT)], axis=1)
        cnt = c_ref[0, :, 0:1] + c_ref[1, :, 0:1]
        pooled = sums / jnp.clip(cnt, 1.0, None)
        h = lax.dot_general(
            pooled, w1_ref[...],
            dimension_numbers=(((1,), (1,)), ((), ())),
            preferred_element_type=jnp.float32,
            precision=lax.Precision.HIGHEST)
        h = jnp.maximum(h + b1_ref[...], 0.0)
        o = lax.dot_general(
            h, w2_ref[...],
            dimension_numbers=(((1,), (1,)), ((), ())),
            preferred_element_type=jnp.float32,
            precision=lax.Precision.HIGHEST)
        out_ref[...] = o[:, 0:1] + b2_ref[0, 0]

    return pl.pallas_call(
        body,
        in_specs=[
            pl.BlockSpec(memory_space=pltpu.VMEM),
            pl.BlockSpec(memory_space=pltpu.VMEM),
            pl.BlockSpec(memory_space=pltpu.VMEM),
            pl.BlockSpec(memory_space=pltpu.VMEM),
            pl.BlockSpec(memory_space=pltpu.VMEM),
            pl.BlockSpec(memory_space=pltpu.SMEM),
        ],
        out_shape=jax.ShapeDtypeStruct((NSEG, 1), jnp.float32),
    )(sums4, cnt2, W1, b1r, W2, b2r)


@jax.jit
def kernel(x, batch, W1, b1, W2, b2):
    xp = (x.reshape(NGRP, 8, CT, LANES)
          .transpose(0, 2, 1, 3)
          .reshape(NGRP * CT, 8, LANES))
    batch2 = batch.astype(jnp.int32).reshape(NCH, CHUNK)
    ones = jnp.ones((CHUNK, CW), jnp.float32)
    zs = jnp.zeros((ARS, LANES), jnp.float32)
    zc = jnp.zeros((RS, CW), jnp.float32)
    sums2, cnt2 = _sc_segment_sum(xp, batch2, ones, zs, zc)
    sums4 = sums2.reshape(NC, CT, NSEG, LANES)
    W2p = jnp.pad(W2, ((0, 7), (0, 0)))
    out = _tc_head(sums4, cnt2, W1, b1.reshape(1, 128), W2p, b2.reshape(1, 1))
    return out[:, 0]
